# baseline (speedup 1.0000x reference)
import functools

import jax
import jax.numpy as jnp
from jax import lax
from jax.experimental import pallas as pl
from jax.experimental.pallas import tpu as pltpu
from jax.experimental.pallas import tpu_sc as plsc

E = 8
TOP_K = 2
DIM = 1024
DFF = 4096
T = 32
CHUNK = 1024
NJ = DFF // CHUNK
LANES = 16
NC = 2


@functools.partial(
    pl.kernel,
    mesh=plsc.VectorSubcoreMesh(core_axis_name="c", subcore_axis_name="s"),
    out_type=jax.ShapeDtypeStruct((T, LANES), jnp.float32),
    scratch_types=[
        pltpu.VMEM((DIM,), jnp.float32),
        pltpu.VMEM((E, DIM), jnp.float32),
        pltpu.VMEM((LANES,), jnp.float32),
    ],
)
def _sc_gate(x_hbm, wg_hbm, coef_hbm, xv, wgv, coefv):
    wid = lax.axis_index("s") * NC + lax.axis_index("c")
    pltpu.sync_copy(x_hbm.at[wid], xv)
    pltpu.sync_copy(wg_hbm, wgv)

    iota = lax.iota(jnp.int32, LANES)

    def bfly(v, binop):
        for s in (1, 2, 4, 8):
            w = v.at[iota ^ s].get(
                mode=lax.GatherScatterMode.PROMISE_IN_BOUNDS)
            v = binop(v, w)
        return v

    lg = jnp.full((LANES,), -jnp.inf, dtype=jnp.float32)
    for e in range(E):
        acc = jnp.zeros((LANES,), jnp.float32)
        for k in range(DIM // LANES):
            acc = acc + (xv[pl.ds(k * LANES, LANES)]
                         * wgv[e, pl.ds(k * LANES, LANES)])
        lg = jnp.where(iota == e, bfly(acc, jnp.add), lg)

    m1v = bfly(lg, jnp.maximum)
    i1v = bfly(jnp.where(lg == m1v, iota, LANES), jnp.minimum)
    masked = jnp.where(iota == i1v, -jnp.inf, lg)
    m2v = bfly(masked, jnp.maximum)
    i2v = bfly(jnp.where(masked == m2v, iota, LANES), jnp.minimum)
    e2 = jnp.exp(m2v - m1v)
    den = e2 + 1.0
    zero = jnp.zeros((LANES,), jnp.float32)
    coefv[...] = (jnp.where(iota == i1v, 1.0 / den, zero)
                  + jnp.where(iota == i2v, e2 / den, zero))
    pltpu.sync_copy(coefv, coef_hbm.at[wid])


def _stream_body(x_ref, coef_ref, w1_ref, w3_ref, w2_ref, out_ref):
    i = pl.program_id(0)
    j = pl.program_id(1)

    @pl.when((i == 0) & (j == 0))
    def _():
        out_ref[...] = jnp.zeros_like(out_ref)

    x = x_ref[...].astype(jnp.bfloat16)
    a = jax.lax.dot_general(
        x, w1_ref[0].astype(jnp.bfloat16), (((1,), (1,)), ((), ())),
        preferred_element_type=jnp.float32)
    b = jax.lax.dot_general(
        x, w3_ref[0].astype(jnp.bfloat16), (((1,), (1,)), ((), ())),
        preferred_element_type=jnp.float32)
    h = a * jax.lax.logistic(a) * b
    iota = jax.lax.broadcasted_iota(jnp.int32, (T, LANES), 1)
    c = jnp.sum(jnp.where(iota == i, coef_ref[...], 0.0), axis=1,
                keepdims=True)
    h = (h * c).astype(jnp.bfloat16)
    out_ref[...] += jax.lax.dot_general(
        h, w2_ref[0].astype(jnp.bfloat16), (((1,), (1,)), ((), ())),
        preferred_element_type=jnp.float32)


def _stream(x, coef, w1, w3, w2):
    return pl.pallas_call(
        _stream_body,
        grid=(E, NJ),
        in_specs=[
            pl.BlockSpec((T, DIM), lambda i, j: (0, 0)),
            pl.BlockSpec((T, LANES), lambda i, j: (0, 0)),
            pl.BlockSpec((1, CHUNK, DIM), lambda i, j: (i, j, 0)),
            pl.BlockSpec((1, CHUNK, DIM), lambda i, j: (i, j, 0)),
            pl.BlockSpec((1, DIM, CHUNK), lambda i, j: (i, 0, j)),
        ],
        out_specs=pl.BlockSpec((T, DIM), lambda i, j: (0, 0)),
        out_shape=jax.ShapeDtypeStruct((T, DIM), jnp.float32),
    )(x, coef, w1, w3, w2)


def kernel(inputs, Wg, w1, w2, w3):
    x = inputs.reshape(-1, inputs.shape[-1])
    coef = _sc_gate(x, Wg)
    out = _stream(x, coef, w1, w3, w2)
    return out.reshape(inputs.shape)


# speedup vs baseline: 1.0308x; 1.0308x over previous
import functools

import jax
import jax.numpy as jnp
from jax import lax
from jax.experimental import pallas as pl
from jax.experimental.pallas import tpu as pltpu
from jax.experimental.pallas import tpu_sc as plsc

E = 8
TOP_K = 2
DIM = 1024
DFF = 4096
T = 32
CHUNK = 1024
NJ = DFF // CHUNK
LANES = 16
NC = 2


@functools.partial(
    pl.kernel,
    mesh=plsc.VectorSubcoreMesh(core_axis_name="c", subcore_axis_name="s"),
    out_type=jax.ShapeDtypeStruct((T, LANES), jnp.float32),
    scratch_types=[
        pltpu.VMEM((DIM,), jnp.float32),
        pltpu.VMEM((E, DIM), jnp.float32),
        pltpu.VMEM((LANES,), jnp.float32),
        pltpu.SemaphoreType.DMA,
        pltpu.SemaphoreType.DMA,
    ],
)
def _sc_gate(x_hbm, wg_hbm, coef_hbm, xv, wgv, coefv, sem1, sem2):
    wid = lax.axis_index("s") * NC + lax.axis_index("c")
    c1 = pltpu.async_copy(x_hbm.at[wid], xv, sem1)
    c2 = pltpu.async_copy(wg_hbm, wgv, sem2)
    c1.wait()
    c2.wait()

    iota = lax.iota(jnp.int32, LANES)

    def bfly(v, binop):
        for s in (1, 2, 4, 8):
            w = v.at[iota ^ s].get(
                mode=lax.GatherScatterMode.PROMISE_IN_BOUNDS)
            v = binop(v, w)
        return v

    lg = jnp.full((LANES,), -jnp.inf, dtype=jnp.float32)
    for e in range(E):
        acc = jnp.zeros((LANES,), jnp.float32)
        for k in range(DIM // LANES):
            acc = acc + (xv[pl.ds(k * LANES, LANES)]
                         * wgv[e, pl.ds(k * LANES, LANES)])
        lg = jnp.where(iota == e, bfly(acc, jnp.add), lg)

    m1v = bfly(lg, jnp.maximum)
    i1v = bfly(jnp.where(lg == m1v, iota, LANES), jnp.minimum)
    masked = jnp.where(iota == i1v, -jnp.inf, lg)
    m2v = bfly(masked, jnp.maximum)
    i2v = bfly(jnp.where(masked == m2v, iota, LANES), jnp.minimum)
    e2 = jnp.exp(m2v - m1v)
    den = e2 + 1.0
    zero = jnp.zeros((LANES,), jnp.float32)
    coefv[...] = (jnp.where(iota == i1v, 1.0 / den, zero)
                  + jnp.where(iota == i2v, e2 / den, zero))
    pltpu.sync_copy(coefv, coef_hbm.at[wid])


def _stream_body(x_ref, w1_ref, w3_ref, w2_ref, acc_ref):
    i = pl.program_id(0)
    j = pl.program_id(1)

    x = x_ref[...].astype(jnp.bfloat16)
    a = jax.lax.dot_general(
        x, w1_ref[0].astype(jnp.bfloat16), (((1,), (1,)), ((), ())),
        preferred_element_type=jnp.float32)
    b = jax.lax.dot_general(
        x, w3_ref[0].astype(jnp.bfloat16), (((1,), (1,)), ((), ())),
        preferred_element_type=jnp.float32)
    h = (a * jax.lax.logistic(a) * b).astype(jnp.bfloat16)
    val = jax.lax.dot_general(
        h, w2_ref[0].astype(jnp.bfloat16), (((1,), (1,)), ((), ())),
        preferred_element_type=jnp.float32)
    base = i * T

    @pl.when(j == 0)
    def _():
        acc_ref[pl.ds(base, T), :] = val

    @pl.when(j > 0)
    def _():
        acc_ref[pl.ds(base, T), :] += val


def _stream(x, w1, w3, w2):
    return pl.pallas_call(
        _stream_body,
        grid=(E, NJ),
        in_specs=[
            pl.BlockSpec((T, DIM), lambda i, j: (0, 0)),
            pl.BlockSpec((1, CHUNK, DIM), lambda i, j: (i, j, 0)),
            pl.BlockSpec((1, CHUNK, DIM), lambda i, j: (i, j, 0)),
            pl.BlockSpec((1, DIM, CHUNK), lambda i, j: (i, 0, j)),
        ],
        out_specs=pl.BlockSpec((E * T, DIM), lambda i, j: (0, 0)),
        out_shape=jax.ShapeDtypeStruct((E * T, DIM), jnp.float32),
    )(x, w1, w3, w2)


def _combine_body(acc_ref, coef_ref, out_ref):
    coef = coef_ref[...]
    out = jnp.zeros((T, DIM), jnp.float32)
    for e in range(E):
        out = out + coef[:, e][:, None] * acc_ref[e * T:(e + 1) * T, :]
    out_ref[...] = out


def _combine(acc, coef):
    return pl.pallas_call(
        _combine_body,
        out_shape=jax.ShapeDtypeStruct((T, DIM), jnp.float32),
    )(acc, coef)


def kernel(inputs, Wg, w1, w2, w3):
    x = inputs.reshape(-1, inputs.shape[-1])
    coef = _sc_gate(x, Wg)
    acc = _stream(x, w1, w3, w2)
    out = _combine(acc, coef)
    return out.reshape(inputs.shape)


# speedup vs baseline: 1.0343x; 1.0034x over previous
import functools

import jax
import jax.numpy as jnp
from jax import lax
from jax.experimental import pallas as pl
from jax.experimental.pallas import tpu as pltpu
from jax.experimental.pallas import tpu_sc as plsc

E = 8
TOP_K = 2
DIM = 1024
DFF = 4096
T = 32
CHUNK = 1024
NJ = DFF // CHUNK
LANES = 16
NC = 2


def _gate_body(x_ref, wg_ref, out_ref):
    lgt = jax.lax.dot_general(
        x_ref[...], wg_ref[...], (((1,), (1,)), ((), ())),
        preferred_element_type=jnp.float32)
    pad = jnp.full((T, LANES - E), -jnp.inf, jnp.float32)
    out_ref[...] = jnp.concatenate([lgt, pad], axis=1)


def _tc_gate(x, Wg):
    return pl.pallas_call(
        _gate_body,
        out_shape=jax.ShapeDtypeStruct((T, LANES), jnp.float32),
    )(x, Wg)


@functools.partial(
    pl.kernel,
    mesh=plsc.VectorSubcoreMesh(core_axis_name="c", subcore_axis_name="s"),
    out_type=jax.ShapeDtypeStruct((T, LANES), jnp.float32),
    scratch_types=[
        pltpu.VMEM((LANES,), jnp.float32),
        pltpu.VMEM((LANES,), jnp.float32),
    ],
)
def _sc_route(lg_hbm, coef_hbm, lgv, coefv):
    wid = lax.axis_index("s") * NC + lax.axis_index("c")
    pltpu.sync_copy(lg_hbm.at[wid], lgv)
    lg = lgv[...]

    iota = lax.iota(jnp.int32, LANES)

    def bfly(v, binop):
        for s in (1, 2, 4, 8):
            w = v.at[iota ^ s].get(
                mode=lax.GatherScatterMode.PROMISE_IN_BOUNDS)
            v = binop(v, w)
        return v

    m1v = bfly(lg, jnp.maximum)
    i1v = bfly(jnp.where(lg == m1v, iota, LANES), jnp.minimum)
    masked = jnp.where(iota == i1v, -jnp.inf, lg)
    m2v = bfly(masked, jnp.maximum)
    i2v = bfly(jnp.where(masked == m2v, iota, LANES), jnp.minimum)
    e2 = jnp.exp(m2v - m1v)
    den = e2 + 1.0
    zero = jnp.zeros((LANES,), jnp.float32)
    coefv[...] = (jnp.where(iota == i1v, 1.0 / den, zero)
                  + jnp.where(iota == i2v, e2 / den, zero))
    pltpu.sync_copy(coefv, coef_hbm.at[wid])


def _stream_body(x_ref, coef_ref, w1_ref, w3_ref, w2_ref, out_ref):
    i = pl.program_id(0)
    j = pl.program_id(1)

    @pl.when((i == 0) & (j == 0))
    def _():
        out_ref[...] = jnp.zeros_like(out_ref)

    x = x_ref[...].astype(jnp.bfloat16)
    a = jax.lax.dot_general(
        x, w1_ref[0].astype(jnp.bfloat16), (((1,), (1,)), ((), ())),
        preferred_element_type=jnp.float32)
    b = jax.lax.dot_general(
        x, w3_ref[0].astype(jnp.bfloat16), (((1,), (1,)), ((), ())),
        preferred_element_type=jnp.float32)
    h = a * jax.lax.logistic(a) * b
    iota = jax.lax.broadcasted_iota(jnp.int32, (T, LANES), 1)
    c = jnp.sum(jnp.where(iota == i, coef_ref[...], 0.0), axis=1,
                keepdims=True)
    h = (h * c).astype(jnp.bfloat16)
    out_ref[...] += jax.lax.dot_general(
        h, w2_ref[0].astype(jnp.bfloat16), (((1,), (1,)), ((), ())),
        preferred_element_type=jnp.float32)


def _stream(x, coef, w1, w3, w2):
    return pl.pallas_call(
        _stream_body,
        grid=(E, NJ),
        in_specs=[
            pl.BlockSpec((T, DIM), lambda i, j: (0, 0)),
            pl.BlockSpec((T, LANES), lambda i, j: (0, 0)),
            pl.BlockSpec((1, CHUNK, DIM), lambda i, j: (i, j, 0)),
            pl.BlockSpec((1, CHUNK, DIM), lambda i, j: (i, j, 0)),
            pl.BlockSpec((1, DIM, CHUNK), lambda i, j: (i, 0, j)),
        ],
        out_specs=pl.BlockSpec((T, DIM), lambda i, j: (0, 0)),
        out_shape=jax.ShapeDtypeStruct((T, DIM), jnp.float32),
    )(x, coef, w1, w3, w2)


def kernel(inputs, Wg, w1, w2, w3):
    x = inputs.reshape(-1, inputs.shape[-1])
    lg = _tc_gate(x, Wg)
    coef = _sc_route(lg)
    out = _stream(x, coef, w1, w3, w2)
    return out.reshape(inputs.shape)


# speedup vs baseline: 1.0448x; 1.0102x over previous
import functools

import jax
import jax.numpy as jnp
from jax import lax
from jax.experimental import pallas as pl
from jax.experimental.pallas import tpu as pltpu
from jax.experimental.pallas import tpu_sc as plsc

E = 8
TOP_K = 2
DIM = 1024
DFF = 4096
T = 32
CHUNK = 1024
NJ = DFF // CHUNK
LANES = 16
NC = 2


@functools.partial(
    pl.kernel,
    mesh=plsc.VectorSubcoreMesh(core_axis_name="c", subcore_axis_name="s"),
    out_type=jax.ShapeDtypeStruct((T, LANES), jnp.float32),
    scratch_types=[
        pltpu.VMEM((DIM,), jnp.float32),
        pltpu.VMEM((E, DIM), jnp.float32),
        pltpu.VMEM((LANES,), jnp.float32),
    ],
)
def _sc_gate(x_hbm, wg_hbm, coef_hbm, xv, wgv, coefv):
    wid = lax.axis_index("s") * NC + lax.axis_index("c")
    pltpu.sync_copy(x_hbm.at[wid], xv)
    pltpu.sync_copy(wg_hbm, wgv)

    iota = lax.iota(jnp.int32, LANES)

    def bfly(v, binop):
        for s in (1, 2, 4, 8):
            w = v.at[iota ^ s].get(
                mode=lax.GatherScatterMode.PROMISE_IN_BOUNDS)
            v = binop(v, w)
        return v

    lg = jnp.full((LANES,), -jnp.inf, dtype=jnp.float32)
    for e in range(E):
        acc = jnp.zeros((LANES,), jnp.float32)
        for k in range(DIM // LANES):
            acc = acc + (xv[pl.ds(k * LANES, LANES)]
                         * wgv[e, pl.ds(k * LANES, LANES)])
        lg = jnp.where(iota == e, bfly(acc, jnp.add), lg)

    m1v = bfly(lg, jnp.maximum)
    i1v = bfly(jnp.where(lg == m1v, iota, LANES), jnp.minimum)
    masked = jnp.where(iota == i1v, -jnp.inf, lg)
    m2v = bfly(masked, jnp.maximum)
    i2v = bfly(jnp.where(masked == m2v, iota, LANES), jnp.minimum)
    e2 = jnp.exp(m2v - m1v)
    den = e2 + 1.0
    zero = jnp.zeros((LANES,), jnp.float32)
    coefv[...] = (jnp.where(iota == i1v, 1.0 / den, zero)
                  + jnp.where(iota == i2v, e2 / den, zero))
    pltpu.sync_copy(coefv, coef_hbm.at[wid])


def _stream_body(x_ref, w1_ref, w3_ref, w2_ref, acc_ref):
    j = pl.program_id(1)

    @pl.when(j == 0)
    def _():
        acc_ref[...] = jnp.zeros_like(acc_ref)

    x = x_ref[...].astype(jnp.bfloat16)
    a = jax.lax.dot_general(
        x, w1_ref[0].astype(jnp.bfloat16), (((1,), (1,)), ((), ())),
        preferred_element_type=jnp.float32)
    b = jax.lax.dot_general(
        x, w3_ref[0].astype(jnp.bfloat16), (((1,), (1,)), ((), ())),
        preferred_element_type=jnp.float32)
    h = (a * jax.lax.logistic(a) * b).astype(jnp.bfloat16)
    acc_ref[0] += jax.lax.dot_general(
        h, w2_ref[0].astype(jnp.bfloat16), (((1,), (1,)), ((), ())),
        preferred_element_type=jnp.float32)


def _stream(x, w1, w3, w2):
    return pl.pallas_call(
        _stream_body,
        grid=(E, NJ),
        in_specs=[
            pl.BlockSpec((T, DIM), lambda i, j: (0, 0)),
            pl.BlockSpec((1, CHUNK, DIM), lambda i, j: (i, j, 0)),
            pl.BlockSpec((1, CHUNK, DIM), lambda i, j: (i, j, 0)),
            pl.BlockSpec((1, DIM, CHUNK), lambda i, j: (i, 0, j)),
        ],
        out_specs=pl.BlockSpec((1, T, DIM), lambda i, j: (i, 0, 0)),
        out_shape=jax.ShapeDtypeStruct((E, T, DIM), jnp.float32),
    )(x, w1, w3, w2)


def _combine_body(acc_ref, coef_ref, out_ref):
    coef = coef_ref[...]
    out = jnp.zeros((T, DIM), jnp.float32)
    for e in range(E):
        out = out + coef[:, e][:, None] * acc_ref[e]
    out_ref[...] = out


def _combine(acc, coef):
    return pl.pallas_call(
        _combine_body,
        out_shape=jax.ShapeDtypeStruct((T, DIM), jnp.float32),
    )(acc, coef)


def kernel(inputs, Wg, w1, w2, w3):
    x = inputs.reshape(-1, inputs.shape[-1])
    coef = _sc_gate(x, Wg)
    acc = _stream(x, w1, w3, w2)
    out = _combine(acc, coef)
    return out.reshape(inputs.shape)
